# Initial kernel scaffold; baseline (speedup 1.0000x reference)
import jax
import jax.numpy as jnp
from jax.experimental import pallas as pl


def kernel(x, weight):
    raise NotImplementedError("write your pallas kernel here")


# speedup vs baseline: 1.1570x; 1.1570x over previous
import functools

import jax
import jax.numpy as jnp
from jax import lax
from jax.experimental import pallas as pl
from jax.experimental.pallas import tpu as pltpu
from jax.experimental.pallas import tpu_sc as plsc

_B = 16384 * 50
_D = 32
_NUM_CORES = 2
_NUM_SUBCORES = 16
_NW = _NUM_CORES * _NUM_SUBCORES
_B_PER_W = _B // _NW
_CHUNK = 1600
_NCHUNK = _B_PER_W // _CHUNK


@functools.partial(
    pl.kernel,
    mesh=plsc.VectorSubcoreMesh(core_axis_name="c", subcore_axis_name="s"),
    out_type=jax.ShapeDtypeStruct((_B, _D), jnp.float32),
    scratch_types=[
        pltpu.VMEM((_B_PER_W,), jnp.int32),
        pltpu.VMEM((_CHUNK, _D), jnp.float32),
        pltpu.SemaphoreType.DMA,
    ],
    compiler_params=pltpu.CompilerParams(use_tc_tiling_on_sc=False),
)
def _gather_kernel(x_hbm, w_hbm, out_hbm, idx_v, rows_v, sem):
    wid = lax.axis_index("s") * _NUM_CORES + lax.axis_index("c")
    base = wid * _B_PER_W
    pltpu.sync_copy(x_hbm.at[pl.ds(base, _B_PER_W)], idx_v)

    def body(i, carry):
        off = pl.multiple_of(i * _CHUNK, 8)
        pltpu.async_copy(w_hbm.at[idx_v.at[pl.ds(off, _CHUNK)]], rows_v, sem).wait()
        pltpu.sync_copy(rows_v, out_hbm.at[pl.ds(base + off, _CHUNK)])
        return carry

    lax.fori_loop(0, _NCHUNK, body, 0)


def kernel(x, weight):
    flat = _gather_kernel(x.reshape(-1).astype(jnp.int32), weight)
    return flat.reshape(*x.shape, _D)


# speedup vs baseline: 1.1629x; 1.0051x over previous
import functools

import jax
import jax.numpy as jnp
from jax import lax
from jax.experimental import pallas as pl
from jax.experimental.pallas import tpu as pltpu
from jax.experimental.pallas import tpu_sc as plsc

_B = 16384 * 50
_D = 32
_NUM_CORES = 2
_NUM_SUBCORES = 16
_NW = _NUM_CORES * _NUM_SUBCORES
_B_PER_W = _B // _NW
_CHUNK = 1600
_NCHUNK = _B_PER_W // _CHUNK


@functools.partial(
    pl.kernel,
    mesh=plsc.VectorSubcoreMesh(core_axis_name="c", subcore_axis_name="s"),
    out_type=jax.ShapeDtypeStruct((_B, _D), jnp.float32),
    scratch_types=[
        pltpu.VMEM((_B_PER_W,), jnp.int32),
        pltpu.VMEM((_CHUNK, _D), jnp.float32),
        pltpu.VMEM((_CHUNK, _D), jnp.float32),
        pltpu.SemaphoreType.DMA,
        pltpu.SemaphoreType.DMA,
        pltpu.SemaphoreType.DMA,
        pltpu.SemaphoreType.DMA,
    ],
    compiler_params=pltpu.CompilerParams(use_tc_tiling_on_sc=False),
)
def _gather_kernel(x_hbm, w_hbm, out_hbm, idx_v, rows0, rows1, g0, g1, o0, o1):
    wid = lax.axis_index("s") * _NUM_CORES + lax.axis_index("c")
    base = wid * _B_PER_W
    pltpu.sync_copy(x_hbm.at[pl.ds(base, _B_PER_W)], idx_v)

    rows = (rows0, rows1)
    gsem = (g0, g1)
    osem = (o0, o1)

    def gather(i):
        off = i * _CHUNK
        return pltpu.async_copy(
            w_hbm.at[idx_v.at[pl.ds(off, _CHUNK)]], rows[i % 2], gsem[i % 2]
        )

    def write(i):
        off = i * _CHUNK
        return pltpu.async_copy(
            rows[i % 2], out_hbm.at[pl.ds(base + off, _CHUNK)], osem[i % 2]
        )

    pending_g = [None, None]
    pending_o = [None, None]
    pending_g[0] = gather(0)
    for i in range(_NCHUNK):
        b = i % 2
        if i + 1 < _NCHUNK:
            nb = (i + 1) % 2
            if pending_o[nb] is not None:
                pending_o[nb].wait()
                pending_o[nb] = None
            pending_g[nb] = gather(i + 1)
        pending_g[b].wait()
        pending_o[b] = write(i)
    for b in range(2):
        if pending_o[b] is not None:
            pending_o[b].wait()


def kernel(x, weight):
    flat = _gather_kernel(x.reshape(-1).astype(jnp.int32), weight)
    return flat.reshape(*x.shape, _D)


# speedup vs baseline: 1.5077x; 1.2965x over previous
import functools

import jax
import jax.numpy as jnp
from jax import lax
from jax.experimental import pallas as pl
from jax.experimental.pallas import tpu as pltpu
from jax.experimental.pallas import tpu_sc as plsc

_B = 16384
_S = 50
_D = 32
_NUM_CORES = 2
_NUM_SUBCORES = 16
_NW = _NUM_CORES * _NUM_SUBCORES
_B_PER_W = _B // _NW
_BT_PER_W = _B_PER_W // 128
_NBLK = _S * _BT_PER_W


@functools.partial(
    pl.kernel,
    mesh=plsc.VectorSubcoreMesh(core_axis_name="c", subcore_axis_name="s"),
    out_type=jax.ShapeDtypeStruct((_S, _D // 8, _B // 128, 8, 128), jnp.float32),
    scratch_types=[
        pltpu.VMEM((_B_PER_W * _S,), jnp.int32),
        pltpu.VMEM((128,), jnp.int32),
        pltpu.VMEM((128,), jnp.int32),
        pltpu.VMEM((128, _D), jnp.float32),
        pltpu.VMEM((128, _D), jnp.float32),
        pltpu.VMEM((_D // 8, 8, 128), jnp.float32),
        pltpu.VMEM((_D // 8, 8, 128), jnp.float32),
        pltpu.SemaphoreType.DMA,
        pltpu.SemaphoreType.DMA,
    ],
    compiler_params=pltpu.CompilerParams(
        use_tc_tiling_on_sc=False, needs_layout_passes=False),
)
def _gather_kernel(x_hbm, w_hbm, out_hbm, xv, idxv0, idxv1, rows0, rows1,
                   tout0, tout1, g0, g1):
    wid = lax.axis_index("s") * _NUM_CORES + lax.axis_index("c")
    pltpu.sync_copy(x_hbm.at[pl.ds(wid * (_B_PER_W * _S), _B_PER_W * _S)], xv)

    iota = lax.iota(jnp.int32, 16)
    iota_s = iota * _S
    riota = [iota + 16 * c for c in range(8)]

    idxv = (idxv0, idxv1)
    rows = (rows0, rows1)
    tout = (tout0, tout1)
    gsem = (g0, g1)

    def extract_and_fire(a, p):
        s = a // _BT_PER_W
        btl = lax.rem(a, _BT_PER_W)
        base = btl * (128 * _S) + s
        for c in range(8):
            pos = jnp.broadcast_to(base + c * (16 * _S), (16,)) + iota_s
            idxv[p][pl.ds(c * 16, 16)] = plsc.load_gather(xv, [pos])
        return pltpu.async_copy(w_hbm.at[idxv[p]], rows[p], gsem[p])

    def wait_g(p):
        pltpu.make_async_copy(w_hbm.at[idxv[p]], rows[p], gsem[p]).wait()

    def transpose_and_write(a, p):
        s = a // _BT_PER_W
        btg = wid * _BT_PER_W + lax.rem(a, _BT_PER_W)
        for dt in range(_D // 8):
            for dd in range(8):
                col = jnp.broadcast_to(jnp.int32(dt * 8 + dd), (16,))
                for c in range(8):
                    tout[p][dt, dd, pl.ds(c * 16, 16)] = plsc.load_gather(
                        rows[p], [riota[c], col])
        for dt in range(_D // 8):
            pltpu.sync_copy(tout[p].at[dt], out_hbm.at[s, dt, btg])

    extract_and_fire(jnp.int32(0), 0)

    def body(k2, carry):
        a = k2 * 2
        extract_and_fire(a + 1, 1)
        wait_g(0)
        transpose_and_write(a, 0)

        @pl.when(k2 < _NBLK // 2 - 1)
        def _():
            extract_and_fire(a + 2, 0)

        wait_g(1)
        transpose_and_write(a + 1, 1)
        return carry

    lax.fori_loop(0, _NBLK // 2, body, 0)


def kernel(x, weight):
    out5 = _gather_kernel(x.reshape(-1).astype(jnp.int32), weight)
    t = out5.transpose(2, 4, 0, 1, 3)
    return t.reshape(_B, _S, _D)


def _dbg():
    try:
        h = jax.jit(kernel).lower(
            jax.ShapeDtypeStruct((16384, 50), jnp.int32),
            jax.ShapeDtypeStruct((1000000, 32), jnp.float32),
        ).compile().as_text()
        import re
        print("DBG-HLO-LINES", len(h.splitlines()))
        for ln in h.splitlines():
            if re.search(r"call-start|ENTRY|ROOT|copy|bitcast|custom-call", ln):
                print("DBG|", ln.strip()[:250])
    except Exception as e:
        print("DBG-FAIL", repr(e))


import os as _os
if _os.environ.get("K_DBG"):
    _dbg()


# speedup vs baseline: 1.5751x; 1.0447x over previous
import functools

import jax
import jax.numpy as jnp
from jax import lax
from jax.experimental import pallas as pl
from jax.experimental.pallas import tpu as pltpu
from jax.experimental.pallas import tpu_sc as plsc

_B = 16384
_S = 50
_D = 32
_NUM_CORES = 2
_NUM_SUBCORES = 16
_NW = _NUM_CORES * _NUM_SUBCORES
_B_PER_W = _B // _NW
_N_IDX = _B_PER_W * _S
_BLK = 256
_NBLK = _N_IDX // _BLK


@functools.partial(
    pl.kernel,
    mesh=plsc.VectorSubcoreMesh(core_axis_name="c", subcore_axis_name="s"),
    out_type=jax.ShapeDtypeStruct((_S, _D // 8, _B // 128, 8, 128), jnp.float32),
    scratch_types=[
        pltpu.VMEM((_N_IDX,), jnp.int32),
        pltpu.VMEM((_N_IDX,), jnp.int32),
        pltpu.VMEM((_BLK, _D), jnp.float32),
        pltpu.VMEM((_BLK, _D), jnp.float32),
        pltpu.VMEM((_D // 8, 2, 8, 128), jnp.float32),
        pltpu.VMEM((_D // 8, 2, 8, 128), jnp.float32),
        pltpu.SemaphoreType.DMA,
        pltpu.SemaphoreType.DMA,
        pltpu.SemaphoreType.DMA,
        pltpu.SemaphoreType.DMA,
    ],
    compiler_params=pltpu.CompilerParams(
        use_tc_tiling_on_sc=False, needs_layout_passes=False),
)
def _gather_kernel(x_hbm, w_hbm, out_hbm, xv, idx_all, rows0, rows1,
                   tout0, tout1, g0, g1, o0, o1):
    wid = lax.axis_index("s") * _NUM_CORES + lax.axis_index("c")
    pltpu.sync_copy(x_hbm.at[pl.ds(wid * _N_IDX, _N_IDX)], xv)

    iota = lax.iota(jnp.int32, 16)
    iota_s = iota * _S
    riota = [iota + 16 * c for c in range(16)]

    rows = (rows0, rows1)
    tout = (tout0, tout1)
    gsem = (g0, g1)
    osem = (o0, o1)

    def reorder(a, carry):
        s = a // 4
        btl = lax.rem(a, 4)
        base = btl * (128 * _S) + s
        for c in range(8):
            pos = jnp.broadcast_to(base + c * (16 * _S), (16,)) + iota_s
            idx_all[pl.ds(a * 128 + c * 16, 16)] = plsc.load_gather(xv, [pos])
        return carry

    lax.fori_loop(0, _S * 4, reorder, 0)

    def fire_gather(a, p):
        return pltpu.async_copy(
            w_hbm.at[idx_all.at[pl.ds(a * _BLK, _BLK)]], rows[p], gsem[p])

    def wait_gather(a, p):
        pltpu.make_async_copy(
            w_hbm.at[idx_all.at[pl.ds(a * _BLK, _BLK)]], rows[p], gsem[p]).wait()

    def out_slices(a, p):
        s = a // 2
        btg0 = wid * 4 + lax.rem(a, 2) * 2
        return [(tout[p].at[dt], out_hbm.at[s, dt, pl.ds(btg0, 2)])
                for dt in range(_D // 8)]

    def fire_writes(a, p):
        for src, dst in out_slices(a, p):
            pltpu.async_copy(src, dst, osem[p])

    def wait_writes(a, p):
        for src, dst in out_slices(a, p):
            pltpu.make_async_copy(src, dst, osem[p]).wait()

    def transpose(p):
        for btl2 in range(2):
            for c in range(8):
                rowv = riota[btl2 * 8 + c]
                for d in range(_D):
                    colv = jnp.broadcast_to(jnp.int32(d), (16,))
                    tout[p][d // 8, btl2, d % 8,
                            pl.ds(c * 16, 16)] = plsc.load_gather(
                                rows[p], [rowv, colv])

    fire_gather(jnp.int32(0), 0)

    def body(k, carry):
        a = k * 2
        fire_gather(a + 1, 1)
        wait_gather(a, 0)

        @pl.when(k > 0)
        def _():
            wait_writes(a - 2, 0)

        transpose(0)
        fire_writes(a, 0)

        @pl.when(k < _NBLK // 2 - 1)
        def _():
            fire_gather(a + 2, 0)

        wait_gather(a + 1, 1)

        @pl.when(k > 0)
        def _():
            wait_writes(a - 1, 1)

        transpose(1)
        fire_writes(a + 1, 1)
        return carry

    lax.fori_loop(0, _NBLK // 2, body, 0)
    wait_writes(jnp.int32(_NBLK - 2), 0)
    wait_writes(jnp.int32(_NBLK - 1), 1)


def kernel(x, weight):
    out5 = _gather_kernel(x.reshape(-1).astype(jnp.int32), weight)
    t = out5.transpose(2, 4, 0, 1, 3)
    return t.reshape(_B, _S, _D)


# speedup vs baseline: 2.1172x; 1.3441x over previous
import functools

import jax
import jax.numpy as jnp
from jax import lax
from jax.experimental import pallas as pl
from jax.experimental.pallas import tpu as pltpu
from jax.experimental.pallas import tpu_sc as plsc

_B = 16384
_S = 50
_D = 32
_NUM_CORES = 2
_NUM_SUBCORES = 16
_NW = _NUM_CORES * _NUM_SUBCORES
_B_PER_W = _B // _NW
_N_IDX = _B_PER_W * _S
_BLK = 256
_NBLK = _N_IDX // _BLK


@functools.partial(
    pl.kernel,
    mesh=plsc.VectorSubcoreMesh(core_axis_name="c", subcore_axis_name="s"),
    out_type=jax.ShapeDtypeStruct((_S, _D // 8, _B // 128, 8, 128), jnp.float32),
    scratch_types=[
        pltpu.VMEM((_N_IDX,), jnp.int32),
        pltpu.VMEM((_N_IDX,), jnp.int32),
        pltpu.VMEM((_BLK, _D), jnp.float32),
        pltpu.VMEM((_BLK, _D), jnp.float32),
        pltpu.VMEM((_BLK * (_D + 1),), jnp.float32),
        pltpu.VMEM((_BLK * (_D + 1),), jnp.float32),
        pltpu.VMEM((_D // 8, 2, 8, 128), jnp.float32),
        pltpu.VMEM((_D // 8, 2, 8, 128), jnp.float32),
        pltpu.SemaphoreType.DMA,
        pltpu.SemaphoreType.DMA,
        pltpu.SemaphoreType.DMA,
        pltpu.SemaphoreType.DMA,
    ],
    compiler_params=pltpu.CompilerParams(
        use_tc_tiling_on_sc=False, needs_layout_passes=False),
)
def _gather_kernel(x_hbm, w_hbm, out_hbm, xv, idx_all, rows0, rows1,
                   rpad0, rpad1, tout0, tout1, g0, g1, o0, o1):
    wid = lax.axis_index("s") * _NUM_CORES + lax.axis_index("c")
    pltpu.sync_copy(x_hbm.at[pl.ds(wid * _N_IDX, _N_IDX)], xv)

    iota = lax.iota(jnp.int32, 16)
    iota_s = iota * _S
    riota = [iota + 16 * c for c in range(16)]

    rows = (rows0, rows1)
    rpad = (rpad0, rpad1)
    tout = (tout0, tout1)
    gsem = (g0, g1)
    osem = (o0, o1)

    def reorder(a, carry):
        s = a // 4
        btl = lax.rem(a, 4)
        base = btl * (128 * _S) + s
        for c in range(8):
            pos = jnp.broadcast_to(base + c * (16 * _S), (16,)) + iota_s
            idx_all[pl.ds(a * 128 + c * 16, 16)] = plsc.load_gather(xv, [pos])
        return carry

    lax.fori_loop(0, _S * 4, reorder, 0)

    def fire_gather(a, p):
        return pltpu.async_copy(
            w_hbm.at[idx_all.at[pl.ds(a * _BLK, _BLK)]], rows[p], gsem[p])

    def wait_gather(a, p):
        pltpu.make_async_copy(
            w_hbm.at[idx_all.at[pl.ds(a * _BLK, _BLK)]], rows[p], gsem[p]).wait()

    def out_slices(a, p):
        s = a // 2
        btg0 = wid * 4 + lax.rem(a, 2) * 2
        return [(tout[p].at[dt], out_hbm.at[s, dt, pl.ds(btg0, 2)])
                for dt in range(_D // 8)]

    def fire_writes(a, p):
        for src, dst in out_slices(a, p):
            pltpu.async_copy(src, dst, osem[p])

    def wait_writes(a, p):
        for src, dst in out_slices(a, p):
            pltpu.make_async_copy(src, dst, osem[p]).wait()

    iota33 = iota * (_D + 1)

    def transpose(p):
        def pad_chunk(k64, carry):
            v0, v1 = carry
            base = k64 * 64
            for rl in range(64):
                plsc.store_scatter(rpad[p], [v0],
                                   rows[p][base + rl, pl.ds(0, 16)])
                plsc.store_scatter(rpad[p], [v1],
                                   rows[p][base + rl, pl.ds(16, 16)])
                v0 = v0 + (_D + 1)
                v1 = v1 + (_D + 1)
            return v0, v1

        lax.fori_loop(0, _BLK // 64, pad_chunk, (iota, iota + 16))
        for btl2 in range(2):
            for c in range(8):
                vbs = [iota33 + ((btl2 * 128 + c * 16) * (_D + 1) + m)
                       for m in range(8)]
                for d in range(_D):
                    ref = rpad[p].at[pl.ds((d // 8) * 8,
                                           _BLK * (_D + 1) - (d // 8) * 8)]
                    tout[p][d // 8, btl2, d % 8,
                            pl.ds(c * 16, 16)] = plsc.load_gather(
                                ref, [vbs[d % 8]])

    fire_gather(jnp.int32(0), 0)

    def body(k, carry):
        a = k * 2
        fire_gather(a + 1, 1)
        wait_gather(a, 0)

        @pl.when(k > 0)
        def _():
            wait_writes(a - 2, 0)

        transpose(0)
        fire_writes(a, 0)

        @pl.when(k < _NBLK // 2 - 1)
        def _():
            fire_gather(a + 2, 0)

        wait_gather(a + 1, 1)

        @pl.when(k > 0)
        def _():
            wait_writes(a - 1, 1)

        transpose(1)
        fire_writes(a + 1, 1)
        return carry

    lax.fori_loop(0, _NBLK // 2, body, 0)
    wait_writes(jnp.int32(_NBLK - 2), 0)
    wait_writes(jnp.int32(_NBLK - 1), 1)


def kernel(x, weight):
    out5 = _gather_kernel(x.reshape(-1).astype(jnp.int32), weight)
    t = out5.transpose(2, 4, 0, 1, 3)
    return t.reshape(_B, _S, _D)
